# baseline (speedup 1.0000x reference)
import numpy as np
import jax
import jax.numpy as jnp
from jax.experimental import pallas as pl

_BB = 64


def _toeplitz(Wc, W_in, stride_w, pad_w):
    kh, kw, Cin, Cout = Wc.shape
    W_out = (W_in + 2 * pad_w - kw) // stride_w + 1
    M = np.zeros((kw, W_in, W_out), np.float32)
    for dx in range(kw):
        for wo in range(W_out):
            wi = wo * stride_w + dx - pad_w
            if 0 <= wi < W_in:
                M[dx, wi, wo] = 1.0
    T = jnp.einsum('xab,dxio->daibo', M, Wc)
    return T.reshape(kh, W_in * Cin, W_out * Cout)


def _leaky(x):
    return jnp.maximum(x, jnp.float32(0.01) * x)


def _net(x_ref, T1r, T2r, T3r, T4r, T5r, T6r, Whr,
         rb1, rb2, rb3, rb4, rb5, rb6, rbh, out_ref):
    R = x_ref[...].astype(jnp.bfloat16)
    BB = R.shape[0]

    def mm(a, T):
        H, K = a.shape[1], a.shape[2]
        r = jnp.dot(a.reshape(BB * H, K), T,
                    preferred_element_type=jnp.float32)
        return r.reshape(BB, H, T.shape[1])

    def sdn(a):
        z = jnp.zeros((BB, 1, a.shape[2]), a.dtype)
        return jnp.concatenate([z, a[:, :-1]], axis=1)

    def sup(a):
        z = jnp.zeros((BB, 1, a.shape[2]), a.dtype)
        return jnp.concatenate([a[:, 1:], z], axis=1)

    p = [R[:, c] for c in range(6)]
    z32 = jnp.zeros((BB, 20, 32), R.dtype)
    gm = [jnp.concatenate([pc[:, :, 16 * m: 16 * (m + 1)] for pc in p]
                          + [z32], axis=2)
          for m in range(10)]
    R128 = jnp.concatenate(gm, axis=2)

    Rx = jnp.concatenate(
        [sdn(R128)[:, :, 1152:1280], R128, sup(R128)[:, :, 0:128]],
        axis=2)

    h1 = [_leaky(mm(Rx[:, :, 128 * j: 128 * j + 384], T1r[...]) + rb1[...])
          for j in range(10)]

    h2 = [mm(jnp.concatenate([h1[2 * i], h1[2 * i + 1]], axis=2), T2r[...])
          + rb2[...]
          for i in range(5)]

    def g2(o):
        if o == -1:
            return sdn(h2[4])
        if o == 5:
            return sup(h2[0])
        return h2[o]

    h3 = [_leaky(mm(g2(i - 1), T3r[0]) + mm(g2(i), T3r[1])
                 + mm(g2(i + 1), T3r[2]) + rb3[...])
          for i in range(5)]

    h4 = sum(mm(h3[p], T4r[p]) for p in range(5)) + rb4[...]

    z = jnp.zeros((BB, 1, 64), jnp.float32)
    hp = jnp.concatenate([z, h4, z], axis=1)
    h5 = _leaky(mm(hp[:, 0:20], T5r[0]) + mm(hp[:, 1:21], T5r[1])
                + mm(hp[:, 2:22], T5r[2]) + rb5[...])

    h6 = rb6[...]
    for dy in range(5):
        gd = jnp.concatenate([h5[:, 5 * r + dy: 5 * r + dy + 1, :]
                              for r in range(4)], axis=1)
        h6 = h6 + mm(gd, T6r[dy])

    hf = jnp.concatenate([h6[:, i, :] for i in range(4)], axis=1)
    res = jnp.dot(hf, Whr[...], preferred_element_type=jnp.float32) + rbh[...]
    lv = jnp.clip(res[:, 8:16], -5.0, 0.0)
    out_ref[...] = jnp.concatenate([res[:, 0:8], lv], axis=1)


def kernel(input, W1, b1, W2, b2, W3, b3, W4, b4, W5, b5, W6, b6,
           Wmu, bmu, Wlv, blv):
    B = input.shape[0]
    bf = jnp.bfloat16
    x = input.reshape(B, 6, 20, 160)

    T1 = _toeplitz(W1, 16, 1, 1).reshape(3, 16, 6, 128).transpose(0, 2, 1, 3)
    T1 = jnp.pad(T1, ((0, 0), (0, 2), (0, 0), (0, 0))
                 ).reshape(384, 128).astype(bf)
    T2 = _toeplitz(W2, 16, 2, 0).reshape(256, 64)
    T3 = _toeplitz(W3, 8, 1, 1)
    T4 = _toeplitz(W4, 8, 2, 0)
    T5 = _toeplitz(W5, 4, 1, 1)
    T6 = _toeplitz(W6, 4, 2, 0)

    perm = np.empty(256, np.int32)
    for hh in range(4):
        for ww in range(2):
            for cc in range(32):
                perm[hh * 64 + ww * 32 + cc] = cc * 8 + hh * 2 + ww
    Wh = jnp.zeros((256, 16), jnp.float32)
    Wh = Wh.at[:, 0:7].set(Wmu[:, perm].T).at[:, 8:15].set(Wlv[:, perm].T)
    rbh = jnp.zeros((1, 16), jnp.float32)
    rbh = rbh.at[0, 0:7].set(bmu).at[0, 8:15].set(blv)

    rb = [jnp.tile(b, w)[None, None, :] for b, w in
          ((b1, 16), (b2, 8), (b3, 8), (b4, 4), (b5, 4), (b6, 2))]

    full3 = lambda s: pl.BlockSpec(s, lambda i: (0, 0, 0))
    full2 = lambda s: pl.BlockSpec(s, lambda i: (0, 0))

    out = pl.pallas_call(
        _net,
        grid=(B // _BB,),
        in_specs=[
            pl.BlockSpec((_BB, 6, 20, 160), lambda i: (i, 0, 0, 0)),
            full2((384, 128)), full2((256, 64)), full3((3, 64, 128)),
            full3((5, 128, 64)), full3((3, 64, 128)), full3((5, 128, 64)),
            full2((256, 16)),
            full3((1, 1, 128)), full3((1, 1, 64)), full3((1, 1, 128)),
            full3((1, 1, 64)), full3((1, 1, 128)), full3((1, 1, 64)),
            full2((1, 16)),
        ],
        out_specs=pl.BlockSpec((_BB, 16), lambda i: (i, 0)),
        out_shape=jax.ShapeDtypeStruct((B, 16), jnp.float32),
    )(x, T1, T2, T3, T4, T5, T6, Wh, *rb, rbh)

    return out[:, 0:7], out[:, 8:15]


# speedup vs baseline: 1.1373x; 1.1373x over previous
import numpy as np
import jax
import jax.numpy as jnp
from jax.experimental import pallas as pl

_BB = 32


def _toeplitz(Wc, W_in, stride_w, pad_w):
    kh, kw, Cin, Cout = Wc.shape
    W_out = (W_in + 2 * pad_w - kw) // stride_w + 1
    M = np.zeros((kw, W_in, W_out), np.float32)
    for dx in range(kw):
        for wo in range(W_out):
            wi = wo * stride_w + dx - pad_w
            if 0 <= wi < W_in:
                M[dx, wi, wo] = 1.0
    T = jnp.einsum('xab,dxio->daibo', M, Wc)
    return T.reshape(kh, W_in * Cin, W_out * Cout)


def _leaky(x):
    return jnp.maximum(x, jnp.float32(0.01) * x)


def _net(x_ref, T1r, T2r, T3r, T4r, T5r, T6r, Whr,
         rb1, rb2, rb3, rb4, rb5, rb6, rbh, out_ref):
    R = x_ref[...]
    BB = R.shape[0]

    def mm(a, T):
        H, K = a.shape[1], a.shape[2]
        r = jnp.dot(a.reshape(BB * H, K), T,
                    preferred_element_type=jnp.float32)
        return r.reshape(BB, H, T.shape[1])

    def sdn(a):
        z = jnp.zeros((BB, 1, a.shape[2]), a.dtype)
        return jnp.concatenate([z, a[:, :-1]], axis=1)

    def sup(a):
        z = jnp.zeros((BB, 1, a.shape[2]), a.dtype)
        return jnp.concatenate([a[:, 1:], z], axis=1)

    Rx = jnp.concatenate([sdn(R)[:, :, 864:960], R, sup(R)[:, :, 0:96]],
                         axis=2)

    h1 = [_leaky(mm(Rx[:, :, 96 * j: 96 * j + 288], T1r[...]) + rb1[...])
          for j in range(10)]

    h2 = [mm(h1[2 * i], T2r[0]) + mm(h1[2 * i + 1], T2r[1]) + rb2[...]
          for i in range(5)]

    def g2(o):
        if o == -1:
            return sdn(h2[4])
        if o == 5:
            return sup(h2[0])
        return h2[o]

    h3 = [_leaky(mm(g2(i - 1), T3r[0]) + mm(g2(i), T3r[1])
                 + mm(g2(i + 1), T3r[2]) + rb3[...])
          for i in range(5)]

    h4 = sum(mm(h3[p], T4r[p]) for p in range(5)) + rb4[...]

    z = jnp.zeros((BB, 1, 64), jnp.float32)
    hp = jnp.concatenate([z, h4, z], axis=1)
    h5 = _leaky(mm(hp[:, 0:20], T5r[0]) + mm(hp[:, 1:21], T5r[1])
                + mm(hp[:, 2:22], T5r[2]) + rb5[...])

    h6 = rb6[...]
    for dy in range(5):
        gd = jnp.concatenate([h5[:, 5 * r + dy: 5 * r + dy + 1, :]
                              for r in range(4)], axis=1)
        h6 = h6 + mm(gd, T6r[dy])

    hf = jnp.concatenate([h6[:, i, :] for i in range(4)], axis=1)
    res = jnp.dot(hf, Whr[...], preferred_element_type=jnp.float32) + rbh[...]
    lv = jnp.clip(res[:, 8:16], -5.0, 0.0)
    out_ref[...] = jnp.concatenate([res[:, 0:8], lv], axis=1)


def kernel(input, W1, b1, W2, b2, W3, b3, W4, b4, W5, b5, W6, b6,
           Wmu, bmu, Wlv, blv):
    B = input.shape[0]
    x = (input.reshape(B, 6, 20, 10, 16).transpose(0, 2, 3, 1, 4)
         .reshape(B, 20, 960))

    T1 = (_toeplitz(W1, 16, 1, 1).reshape(3, 16, 6, 128)
          .transpose(0, 2, 1, 3).reshape(288, 128))
    T2 = _toeplitz(W2, 16, 2, 0)
    T3 = _toeplitz(W3, 8, 1, 1)
    T4 = _toeplitz(W4, 8, 2, 0)
    T5 = _toeplitz(W5, 4, 1, 1)
    T6 = _toeplitz(W6, 4, 2, 0)

    perm = np.empty(256, np.int32)
    for hh in range(4):
        for ww in range(2):
            for cc in range(32):
                perm[hh * 64 + ww * 32 + cc] = cc * 8 + hh * 2 + ww
    Wh = jnp.zeros((256, 16), jnp.float32)
    Wh = Wh.at[:, 0:7].set(Wmu[:, perm].T).at[:, 8:15].set(Wlv[:, perm].T)
    rbh = jnp.zeros((1, 16), jnp.float32)
    rbh = rbh.at[0, 0:7].set(bmu).at[0, 8:15].set(blv)

    rb = [jnp.tile(b, w)[None, None, :] for b, w in
          ((b1, 16), (b2, 8), (b3, 8), (b4, 4), (b5, 4), (b6, 2))]

    full3 = lambda s: pl.BlockSpec(s, lambda i: (0, 0, 0))
    full2 = lambda s: pl.BlockSpec(s, lambda i: (0, 0))

    out = pl.pallas_call(
        _net,
        grid=(B // _BB,),
        in_specs=[
            pl.BlockSpec((_BB, 20, 960), lambda i: (i, 0, 0)),
            full2((288, 128)), full3((2, 128, 64)), full3((3, 64, 128)),
            full3((5, 128, 64)), full3((3, 64, 128)), full3((5, 128, 64)),
            full2((256, 16)),
            full3((1, 1, 128)), full3((1, 1, 64)), full3((1, 1, 128)),
            full3((1, 1, 64)), full3((1, 1, 128)), full3((1, 1, 64)),
            full2((1, 16)),
        ],
        out_specs=pl.BlockSpec((_BB, 16), lambda i: (i, 0)),
        out_shape=jax.ShapeDtypeStruct((B, 16), jnp.float32),
    )(x, T1, T2, T3, T4, T5, T6, Wh, *rb, rbh)

    return out[:, 0:7], out[:, 8:15]


# speedup vs baseline: 1.1486x; 1.0099x over previous
import numpy as np
import jax
import jax.numpy as jnp
from jax.experimental import pallas as pl

_BB = 64


def _toeplitz(Wc, W_in, stride_w, pad_w):
    kh, kw, Cin, Cout = Wc.shape
    W_out = (W_in + 2 * pad_w - kw) // stride_w + 1
    M = np.zeros((kw, W_in, W_out), np.float32)
    for dx in range(kw):
        for wo in range(W_out):
            wi = wo * stride_w + dx - pad_w
            if 0 <= wi < W_in:
                M[dx, wi, wo] = 1.0
    T = jnp.einsum('xab,dxio->daibo', M, Wc)
    return T.reshape(kh, W_in * Cin, W_out * Cout)


def _leaky(x):
    return jnp.maximum(x, jnp.float32(0.01) * x)


def _net(x_ref, T1r, T2r, T3r, T4r, T5r, T6r, Whr,
         rb1, rb2, rb3, rb4, rb5, rb6, rbh, out_ref):
    R = x_ref[...]
    BB = R.shape[0]

    def mm(a, T):
        H, K = a.shape[1], a.shape[2]
        r = jnp.dot(a.reshape(BB * H, K), T,
                    preferred_element_type=jnp.float32)
        return r.reshape(BB, H, T.shape[1])

    def sdn(a):
        z = jnp.zeros((BB, 1, a.shape[2]), a.dtype)
        return jnp.concatenate([z, a[:, :-1]], axis=1)

    def sup(a):
        z = jnp.zeros((BB, 1, a.shape[2]), a.dtype)
        return jnp.concatenate([a[:, 1:], z], axis=1)

    Rx = jnp.concatenate([sdn(R)[:, :, 864:960], R, sup(R)[:, :, 0:96]],
                         axis=2)

    h1 = [_leaky(mm(Rx[:, :, 96 * j: 96 * j + 288], T1r[...]) + rb1[...])
          for j in range(10)]

    h2 = [mm(h1[2 * i], T2r[0]) + mm(h1[2 * i + 1], T2r[1]) + rb2[...]
          for i in range(5)]

    def g2(o):
        if o == -1:
            return sdn(h2[4])
        if o == 5:
            return sup(h2[0])
        return h2[o]

    h3 = [_leaky(mm(g2(i - 1), T3r[0]) + mm(g2(i), T3r[1])
                 + mm(g2(i + 1), T3r[2]) + rb3[...])
          for i in range(5)]

    h4 = sum(mm(h3[p], T4r[p]) for p in range(5)) + rb4[...]

    z = jnp.zeros((BB, 1, 64), jnp.float32)
    hp = jnp.concatenate([z, h4, z], axis=1)
    h5 = _leaky(mm(hp[:, 0:20], T5r[0]) + mm(hp[:, 1:21], T5r[1])
                + mm(hp[:, 2:22], T5r[2]) + rb5[...])

    h6 = rb6[...]
    for dy in range(5):
        gd = jnp.concatenate([h5[:, 5 * r + dy: 5 * r + dy + 1, :]
                              for r in range(4)], axis=1)
        h6 = h6 + mm(gd, T6r[dy])

    hf = jnp.concatenate([h6[:, i, :] for i in range(4)], axis=1)
    res = jnp.dot(hf, Whr[...], preferred_element_type=jnp.float32) + rbh[...]
    lv = jnp.clip(res[:, 8:16], -5.0, 0.0)
    out_ref[...] = jnp.concatenate([res[:, 0:8], lv], axis=1)


def kernel(input, W1, b1, W2, b2, W3, b3, W4, b4, W5, b5, W6, b6,
           Wmu, bmu, Wlv, blv):
    B = input.shape[0]
    x = (input.reshape(B, 6, 20, 10, 16).transpose(0, 2, 3, 1, 4)
         .reshape(B, 20, 960))

    T1 = (_toeplitz(W1, 16, 1, 1).reshape(3, 16, 6, 128)
          .transpose(0, 2, 1, 3).reshape(288, 128))
    T2 = _toeplitz(W2, 16, 2, 0)
    T3 = _toeplitz(W3, 8, 1, 1)
    T4 = _toeplitz(W4, 8, 2, 0)
    T5 = _toeplitz(W5, 4, 1, 1)
    T6 = _toeplitz(W6, 4, 2, 0)

    perm = np.empty(256, np.int32)
    for hh in range(4):
        for ww in range(2):
            for cc in range(32):
                perm[hh * 64 + ww * 32 + cc] = cc * 8 + hh * 2 + ww
    Wh = jnp.zeros((256, 16), jnp.float32)
    Wh = Wh.at[:, 0:7].set(Wmu[:, perm].T).at[:, 8:15].set(Wlv[:, perm].T)
    rbh = jnp.zeros((1, 16), jnp.float32)
    rbh = rbh.at[0, 0:7].set(bmu).at[0, 8:15].set(blv)

    rb = [jnp.tile(b, w)[None, None, :] for b, w in
          ((b1, 16), (b2, 8), (b3, 8), (b4, 4), (b5, 4), (b6, 2))]

    full3 = lambda s: pl.BlockSpec(s, lambda i: (0, 0, 0))
    full2 = lambda s: pl.BlockSpec(s, lambda i: (0, 0))

    out = pl.pallas_call(
        _net,
        grid=(B // _BB,),
        in_specs=[
            pl.BlockSpec((_BB, 20, 960), lambda i: (i, 0, 0)),
            full2((288, 128)), full3((2, 128, 64)), full3((3, 64, 128)),
            full3((5, 128, 64)), full3((3, 64, 128)), full3((5, 128, 64)),
            full2((256, 16)),
            full3((1, 1, 128)), full3((1, 1, 64)), full3((1, 1, 128)),
            full3((1, 1, 64)), full3((1, 1, 128)), full3((1, 1, 64)),
            full2((1, 16)),
        ],
        out_specs=pl.BlockSpec((_BB, 16), lambda i: (i, 0)),
        out_shape=jax.ShapeDtypeStruct((B, 16), jnp.float32),
    )(x, T1, T2, T3, T4, T5, T6, Wh, *rb, rbh)

    return out[:, 0:7], out[:, 8:15]
